# baseline (speedup 1.0000x reference)
import functools

import jax
import jax.numpy as jnp
from jax import lax
from jax.experimental import pallas as pl
from jax.experimental.pallas import tpu as pltpu
from jax.experimental.pallas import tpu_sc as plsc

VOCAB_SIZE = 25107
EMB_DIM = 100
CTX_LEN = 5
BATCH = 16
NUM_TOKENS = BATCH * CTX_LEN

_SC_INFO = plsc.get_sparse_core_info()
_NC = _SC_INFO.num_cores
_NS = _SC_INFO.num_subcores
ROWS_PER_WORKER = 8
ACTIVE_WORKERS = NUM_TOKENS // ROWS_PER_WORKER


def _sc_gather(idx_flat, emb):
    mesh = plsc.VectorSubcoreMesh(core_axis_name="c", subcore_axis_name="s")

    @functools.partial(
        pl.kernel,
        mesh=mesh,
        out_type=jax.ShapeDtypeStruct((NUM_TOKENS, EMB_DIM), jnp.float32),
        scratch_types=[
            pltpu.VMEM((16,), jnp.int32),
            pltpu.VMEM((ROWS_PER_WORKER, EMB_DIM), jnp.float32),
            pltpu.SemaphoreType.DMA,
        ],
    )
    def gather_kernel(idx_hbm, table_hbm, out_hbm, idx_v, rows_v, sem):
        wid = lax.axis_index("s") * _NC + lax.axis_index("c")

        @pl.when(wid < ACTIVE_WORKERS)
        def _():
            base = wid * ROWS_PER_WORKER
            pltpu.sync_copy(idx_hbm.at[pl.ds(base, ROWS_PER_WORKER)],
                            idx_v.at[pl.ds(0, ROWS_PER_WORKER)])
            idx_vec = idx_v[...]
            copies = []
            for t in range(ROWS_PER_WORKER):
                row = idx_vec[t]
                copies.append(
                    pltpu.async_copy(table_hbm.at[row], rows_v.at[t], sem))
            for c in copies:
                c.wait()
            pltpu.sync_copy(rows_v, out_hbm.at[pl.ds(base, ROWS_PER_WORKER)])

    return gather_kernel(idx_flat, emb)


VOCAB_TILE = 4096
NUM_VOCAB_TILES = pl.cdiv(VOCAB_SIZE, VOCAB_TILE)


def _mlp_kernel(embedded_ref, w1_ref, b1_ref, w2_ref, b2_ref, w3_ref, b3_ref,
                out_ref, h2_ref):
    @pl.when(pl.program_id(0) == 0)
    def _():
        h1 = jnp.maximum(
            jnp.dot(embedded_ref[...], w1_ref[...],
                    preferred_element_type=jnp.float32) + b1_ref[...], 0.0)
        h2_ref[...] = jnp.maximum(
            jnp.dot(h1, w2_ref[...],
                    preferred_element_type=jnp.float32) + b2_ref[...], 0.0)

    out_ref[...] = jnp.dot(h2_ref[...], w3_ref[...],
                           preferred_element_type=jnp.float32) + b3_ref[...]


def kernel(x, emb, W1, b1, W2, b2, W3, b3):
    embedded = jnp.take(emb, x.reshape(-1), axis=0)
    embedded = embedded.reshape(BATCH, CTX_LEN * EMB_DIM)

    out = pl.pallas_call(
        _mlp_kernel,
        grid=(NUM_VOCAB_TILES,),
        in_specs=[
            pl.BlockSpec((BATCH, CTX_LEN * EMB_DIM), lambda i: (0, 0)),
            pl.BlockSpec((CTX_LEN * EMB_DIM, 300), lambda i: (0, 0)),
            pl.BlockSpec((1, 300), lambda i: (0, 0)),
            pl.BlockSpec((300, 300), lambda i: (0, 0)),
            pl.BlockSpec((1, 300), lambda i: (0, 0)),
            pl.BlockSpec((300, VOCAB_TILE), lambda i: (0, i)),
            pl.BlockSpec((1, VOCAB_TILE), lambda i: (0, i)),
        ],
        out_specs=pl.BlockSpec((BATCH, VOCAB_TILE), lambda i: (0, i)),
        out_shape=jax.ShapeDtypeStruct((BATCH, VOCAB_SIZE), jnp.float32),
        scratch_shapes=[pltpu.VMEM((BATCH, 300), jnp.float32)],
    )(embedded, W1, b1.reshape(1, -1), W2, b2.reshape(1, -1), W3,
      b3.reshape(1, -1))
    return out


# speedup vs baseline: 1.8430x; 1.8430x over previous
import functools

import jax
import jax.numpy as jnp
from jax import lax
from jax.experimental import pallas as pl
from jax.experimental.pallas import tpu as pltpu
from jax.experimental.pallas import tpu_sc as plsc

VOCAB_SIZE = 25107
EMB_DIM = 100
CTX_LEN = 5
BATCH = 16
NUM_TOKENS = BATCH * CTX_LEN

_SC_INFO = plsc.get_sparse_core_info()
_NC = _SC_INFO.num_cores
_NS = _SC_INFO.num_subcores
ROWS_PER_WORKER = 8
ACTIVE_WORKERS = NUM_TOKENS // ROWS_PER_WORKER


def _sc_gather(idx_flat, emb):
    mesh = plsc.VectorSubcoreMesh(core_axis_name="c", subcore_axis_name="s")

    @functools.partial(
        pl.kernel,
        mesh=mesh,
        out_type=jax.ShapeDtypeStruct((NUM_TOKENS, EMB_DIM), jnp.float32),
        scratch_types=[
            pltpu.VMEM((16,), jnp.int32),
            pltpu.VMEM((ROWS_PER_WORKER, EMB_DIM), jnp.float32),
            pltpu.SemaphoreType.DMA,
        ],
    )
    def gather_kernel(idx_hbm, table_hbm, out_hbm, idx_v, rows_v, sem):
        wid = lax.axis_index("s") * _NC + lax.axis_index("c")

        @pl.when(wid < ACTIVE_WORKERS)
        def _():
            base = wid * ROWS_PER_WORKER
            pltpu.sync_copy(idx_hbm.at[pl.ds(base, ROWS_PER_WORKER)],
                            idx_v.at[pl.ds(0, ROWS_PER_WORKER)])
            idx_vec = idx_v[...]
            copies = []
            for t in range(ROWS_PER_WORKER):
                row = idx_vec[t]
                copies.append(
                    pltpu.async_copy(table_hbm.at[row], rows_v.at[t], sem))
            for c in copies:
                c.wait()
            pltpu.sync_copy(rows_v, out_hbm.at[pl.ds(base, ROWS_PER_WORKER)])

    return gather_kernel(idx_flat, emb)


VOCAB_TILE = 4096
NUM_VOCAB_TILES = pl.cdiv(VOCAB_SIZE, VOCAB_TILE)


def _mlp_kernel(embedded_ref, w1_ref, b1_ref, w2_ref, b2_ref, w3_ref, b3_ref,
                out_ref, h2_ref):
    @pl.when(pl.program_id(0) == 0)
    def _():
        h1 = jnp.maximum(
            jnp.dot(embedded_ref[...], w1_ref[...],
                    preferred_element_type=jnp.float32) + b1_ref[...], 0.0)
        h2_ref[...] = jnp.maximum(
            jnp.dot(h1, w2_ref[...],
                    preferred_element_type=jnp.float32) + b2_ref[...], 0.0)

    out_ref[...] = jnp.dot(h2_ref[...].astype(jnp.bfloat16),
                           w3_ref[...].astype(jnp.bfloat16),
                           preferred_element_type=jnp.float32) + b3_ref[...]


def kernel(x, emb, W1, b1, W2, b2, W3, b3):
    embedded = _sc_gather(x.reshape(-1).astype(jnp.int32), emb)
    embedded = embedded.reshape(BATCH, CTX_LEN * EMB_DIM)

    out = pl.pallas_call(
        _mlp_kernel,
        grid=(NUM_VOCAB_TILES,),
        in_specs=[
            pl.BlockSpec((BATCH, CTX_LEN * EMB_DIM), lambda i: (0, 0)),
            pl.BlockSpec((CTX_LEN * EMB_DIM, 300), lambda i: (0, 0)),
            pl.BlockSpec((1, 300), lambda i: (0, 0)),
            pl.BlockSpec((300, 300), lambda i: (0, 0)),
            pl.BlockSpec((1, 300), lambda i: (0, 0)),
            pl.BlockSpec((300, VOCAB_TILE), lambda i: (0, i)),
            pl.BlockSpec((1, VOCAB_TILE), lambda i: (0, i)),
        ],
        out_specs=pl.BlockSpec((BATCH, VOCAB_TILE), lambda i: (0, i)),
        out_shape=jax.ShapeDtypeStruct((BATCH, VOCAB_SIZE), jnp.float32),
        scratch_shapes=[pltpu.VMEM((BATCH, 300), jnp.float32)],
    )(embedded, W1, b1.reshape(1, -1), W2, b2.reshape(1, -1), W3,
      b3.reshape(1, -1))
    return out


# speedup vs baseline: 1.8514x; 1.0046x over previous
import functools

import jax
import jax.numpy as jnp
from jax import lax
from jax.experimental import pallas as pl
from jax.experimental.pallas import tpu as pltpu
from jax.experimental.pallas import tpu_sc as plsc

VOCAB_SIZE = 25107
EMB_DIM = 100
CTX_LEN = 5
BATCH = 16
NUM_TOKENS = BATCH * CTX_LEN

_SC_INFO = plsc.get_sparse_core_info()
_NC = _SC_INFO.num_cores
_NS = _SC_INFO.num_subcores
ROWS_PER_WORKER = 8
ACTIVE_WORKERS = NUM_TOKENS // ROWS_PER_WORKER


def _sc_gather(idx_flat, emb):
    mesh = plsc.VectorSubcoreMesh(core_axis_name="c", subcore_axis_name="s")

    @functools.partial(
        pl.kernel,
        mesh=mesh,
        out_type=jax.ShapeDtypeStruct((NUM_TOKENS, EMB_DIM), jnp.float32),
        scratch_types=[
            pltpu.VMEM((16,), jnp.int32),
            pltpu.VMEM((ROWS_PER_WORKER, EMB_DIM), jnp.float32),
            pltpu.SemaphoreType.DMA,
        ],
    )
    def gather_kernel(idx_hbm, table_hbm, out_hbm, idx_v, rows_v, sem):
        wid = lax.axis_index("s") * _NC + lax.axis_index("c")

        @pl.when(wid < ACTIVE_WORKERS)
        def _():
            base = wid * ROWS_PER_WORKER
            pltpu.sync_copy(idx_hbm.at[pl.ds(base, ROWS_PER_WORKER)],
                            idx_v.at[pl.ds(0, ROWS_PER_WORKER)])
            idx_vec = idx_v[...]
            copies = []
            for t in range(ROWS_PER_WORKER):
                row = idx_vec[t]
                copies.append(
                    pltpu.async_copy(table_hbm.at[row], rows_v.at[t], sem))
            for c in copies:
                c.wait()
            pltpu.sync_copy(rows_v, out_hbm.at[pl.ds(base, ROWS_PER_WORKER)])

    return gather_kernel(idx_flat, emb)


VOCAB_TILE = 2048
NUM_FULL_TILES = VOCAB_SIZE // VOCAB_TILE
TAIL = VOCAB_SIZE - NUM_FULL_TILES * VOCAB_TILE
NBUF = 4


def _mlp_kernel(embedded_ref, w1_ref, b1_ref, w2_ref, b2_ref, w3_hbm, b3_ref,
                out_ref, bufs, tail_buf, sems, tail_sem):
    def start_fetch(i):
        pltpu.make_async_copy(
            w3_hbm.at[:, pl.ds(i * VOCAB_TILE, VOCAB_TILE)],
            bufs.at[i % NBUF],
            sems.at[i % NBUF],
        ).start()

    tail_copy = pltpu.make_async_copy(
        w3_hbm.at[:, pl.ds(NUM_FULL_TILES * VOCAB_TILE, TAIL)],
        tail_buf,
        tail_sem,
    )
    tail_copy.start()
    for i in range(NBUF):
        start_fetch(i)

    h1 = jnp.maximum(
        jnp.dot(embedded_ref[...], w1_ref[...],
                preferred_element_type=jnp.float32) + b1_ref[...], 0.0)
    h2 = jnp.maximum(
        jnp.dot(h1, w2_ref[...],
                preferred_element_type=jnp.float32) + b2_ref[...], 0.0)

    for i in range(NUM_FULL_TILES):
        pltpu.make_async_copy(
            w3_hbm.at[:, pl.ds(i * VOCAB_TILE, VOCAB_TILE)],
            bufs.at[i % NBUF],
            sems.at[i % NBUF],
        ).wait()
        tile = jnp.dot(h2, bufs[i % NBUF],
                       preferred_element_type=jnp.float32)
        if i + NBUF < NUM_FULL_TILES:
            start_fetch(i + NBUF)
        out_ref[:, pl.ds(i * VOCAB_TILE, VOCAB_TILE)] = (
            tile + b3_ref[:, pl.ds(i * VOCAB_TILE, VOCAB_TILE)])

    tail_copy.wait()
    base = NUM_FULL_TILES * VOCAB_TILE
    tail = jnp.dot(h2, tail_buf[...], preferred_element_type=jnp.float32)
    out_ref[:, pl.ds(base, TAIL)] = tail + b3_ref[:, pl.ds(base, TAIL)]


def kernel(x, emb, W1, b1, W2, b2, W3, b3):
    embedded = _sc_gather(x.reshape(-1).astype(jnp.int32), emb)
    embedded = embedded.reshape(BATCH, CTX_LEN * EMB_DIM)

    out = pl.pallas_call(
        _mlp_kernel,
        in_specs=[
            pl.BlockSpec(memory_space=pltpu.VMEM),
            pl.BlockSpec(memory_space=pltpu.VMEM),
            pl.BlockSpec(memory_space=pltpu.VMEM),
            pl.BlockSpec(memory_space=pltpu.VMEM),
            pl.BlockSpec(memory_space=pltpu.VMEM),
            pl.BlockSpec(memory_space=pl.ANY),
            pl.BlockSpec(memory_space=pltpu.VMEM),
        ],
        out_specs=pl.BlockSpec(memory_space=pltpu.VMEM),
        out_shape=jax.ShapeDtypeStruct((BATCH, VOCAB_SIZE), jnp.float32),
        scratch_shapes=[
            pltpu.VMEM((NBUF, 300, VOCAB_TILE), jnp.float32),
            pltpu.VMEM((300, TAIL), jnp.float32),
            pltpu.SemaphoreType.DMA((NBUF,)),
            pltpu.SemaphoreType.DMA,
        ],
    )(embedded, W1, b1.reshape(1, -1), W2, b2.reshape(1, -1), W3,
      b3.reshape(1, -1))
    return out


# speedup vs baseline: 1.8601x; 1.0047x over previous
import functools

import jax
import jax.numpy as jnp
from jax import lax
from jax.experimental import pallas as pl
from jax.experimental.pallas import tpu as pltpu
from jax.experimental.pallas import tpu_sc as plsc

VOCAB_SIZE = 25107
EMB_DIM = 100
CTX_LEN = 5
BATCH = 16
NUM_TOKENS = BATCH * CTX_LEN

_SC_INFO = plsc.get_sparse_core_info()
_NC = _SC_INFO.num_cores
_NS = _SC_INFO.num_subcores
ROWS_PER_WORKER = 8
ACTIVE_WORKERS = NUM_TOKENS // ROWS_PER_WORKER


def _sc_gather(idx_flat, emb):
    mesh = plsc.VectorSubcoreMesh(core_axis_name="c", subcore_axis_name="s")

    @functools.partial(
        pl.kernel,
        mesh=mesh,
        out_type=jax.ShapeDtypeStruct((NUM_TOKENS, EMB_DIM), jnp.float32),
        scratch_types=[
            pltpu.VMEM((16,), jnp.int32),
            pltpu.VMEM((ROWS_PER_WORKER, 8, EMB_DIM), jnp.float32),
            pltpu.VMEM((ROWS_PER_WORKER, EMB_DIM), jnp.float32),
            pltpu.SemaphoreType.DMA,
        ],
    )
    def gather_kernel(idx_hbm, table_hbm, out_hbm, idx_v, tiles_v, rows_v,
                      sem):
        wid = lax.axis_index("s") * _NC + lax.axis_index("c")

        @pl.when(wid < ACTIVE_WORKERS)
        def _():
            base = wid * ROWS_PER_WORKER
            pltpu.sync_copy(idx_hbm.at[pl.ds(base, ROWS_PER_WORKER)],
                            idx_v.at[pl.ds(0, ROWS_PER_WORKER)])
            idx_vec = idx_v[...]
            copies = []
            for t in range(ROWS_PER_WORKER):
                tile_base = (idx_vec[t] // 8) * 8
                copies.append(
                    pltpu.async_copy(
                        table_hbm.at[pl.ds(tile_base, 8)],
                        tiles_v.at[t], sem))
            for c in copies:
                c.wait()
            for t in range(ROWS_PER_WORKER):
                sub = idx_vec[t] % 8
                for off in (0, 16, 32, 48, 64, 80, 84):
                    rows_v[t, pl.ds(off, 16)] = (
                        tiles_v[t, sub, pl.ds(off, 16)])
            pltpu.sync_copy(rows_v, out_hbm.at[pl.ds(base, ROWS_PER_WORKER)])

    return gather_kernel(idx_flat, emb)


VOCAB_TILE = 2048
NUM_FULL_TILES = VOCAB_SIZE // VOCAB_TILE
TAIL = VOCAB_SIZE - NUM_FULL_TILES * VOCAB_TILE
NBUF = 4


def _mlp_kernel(embedded_ref, w1_ref, b1_ref, w2_ref, b2_ref, w3_hbm, b3_ref,
                out_ref, bufs, tail_buf, sems, tail_sem):
    def start_fetch(i):
        pltpu.make_async_copy(
            w3_hbm.at[:, pl.ds(i * VOCAB_TILE, VOCAB_TILE)],
            bufs.at[i % NBUF],
            sems.at[i % NBUF],
        ).start()

    tail_copy = pltpu.make_async_copy(
        w3_hbm.at[:, pl.ds(NUM_FULL_TILES * VOCAB_TILE, TAIL)],
        tail_buf,
        tail_sem,
    )
    tail_copy.start()
    for i in range(NBUF):
        start_fetch(i)

    h1 = jnp.maximum(
        jnp.dot(embedded_ref[...], w1_ref[...],
                preferred_element_type=jnp.float32) + b1_ref[...], 0.0)
    h2 = jnp.maximum(
        jnp.dot(h1, w2_ref[...],
                preferred_element_type=jnp.float32) + b2_ref[...], 0.0)

    for i in range(NUM_FULL_TILES):
        pltpu.make_async_copy(
            w3_hbm.at[:, pl.ds(i * VOCAB_TILE, VOCAB_TILE)],
            bufs.at[i % NBUF],
            sems.at[i % NBUF],
        ).wait()
        tile = jnp.dot(h2, bufs[i % NBUF],
                       preferred_element_type=jnp.float32)
        if i + NBUF < NUM_FULL_TILES:
            start_fetch(i + NBUF)
        out_ref[:, pl.ds(i * VOCAB_TILE, VOCAB_TILE)] = (
            tile + b3_ref[:, pl.ds(i * VOCAB_TILE, VOCAB_TILE)])

    tail_copy.wait()
    base = NUM_FULL_TILES * VOCAB_TILE
    tail = jnp.dot(h2, tail_buf[...], preferred_element_type=jnp.float32)
    out_ref[:, pl.ds(base, TAIL)] = tail + b3_ref[:, pl.ds(base, TAIL)]


def kernel(x, emb, W1, b1, W2, b2, W3, b3):
    embedded = _sc_gather(x.reshape(-1).astype(jnp.int32), emb)
    embedded = embedded.reshape(BATCH, CTX_LEN * EMB_DIM)

    out = pl.pallas_call(
        _mlp_kernel,
        in_specs=[
            pl.BlockSpec(memory_space=pltpu.VMEM),
            pl.BlockSpec(memory_space=pltpu.VMEM),
            pl.BlockSpec(memory_space=pltpu.VMEM),
            pl.BlockSpec(memory_space=pltpu.VMEM),
            pl.BlockSpec(memory_space=pltpu.VMEM),
            pl.BlockSpec(memory_space=pl.ANY),
            pl.BlockSpec(memory_space=pltpu.VMEM),
        ],
        out_specs=pl.BlockSpec(memory_space=pltpu.VMEM),
        out_shape=jax.ShapeDtypeStruct((BATCH, VOCAB_SIZE), jnp.float32),
        scratch_shapes=[
            pltpu.VMEM((NBUF, 300, VOCAB_TILE), jnp.float32),
            pltpu.VMEM((300, TAIL), jnp.float32),
            pltpu.SemaphoreType.DMA((NBUF,)),
            pltpu.SemaphoreType.DMA,
        ],
    )(embedded, W1, b1.reshape(1, -1), W2, b2.reshape(1, -1), W3,
      b3.reshape(1, -1))
    return out


# speedup vs baseline: 2.8479x; 1.5310x over previous
import jax
import jax.numpy as jnp
from jax.experimental import pallas as pl
from jax.experimental.pallas import tpu as pltpu

VOCAB_SIZE = 25107
EMB_DIM = 100
CTX_LEN = 5
BATCH = 16
H1 = 300
H2 = 300

VOCAB_TILE = 2048
NUM_FULL_TILES = VOCAB_SIZE // VOCAB_TILE
TAIL = VOCAB_SIZE - NUM_FULL_TILES * VOCAB_TILE
NBUF = 4


def _mlp_kernel(x_smem, emb_hbm, w1_ref, b1_ref, w2_ref, b2_ref, w3_hbm,
                b3_ref, out_ref, ebuf, bufs, tail_buf, gsem, sems, tail_sem):
    def start_fetch(i):
        pltpu.make_async_copy(
            w3_hbm.at[:, pl.ds(i * VOCAB_TILE, VOCAB_TILE)],
            bufs.at[i % NBUF],
            sems.at[i % NBUF],
        ).start()

    tail_copy = pltpu.make_async_copy(
        w3_hbm.at[:, pl.ds(NUM_FULL_TILES * VOCAB_TILE, TAIL)],
        tail_buf,
        tail_sem,
    )
    tail_copy.start()
    for i in range(NBUF):
        start_fetch(i)

    gathers = []
    for b in range(BATCH):
        for c in range(CTX_LEN):
            g = pltpu.make_async_copy(
                emb_hbm.at[x_smem[b, c]], ebuf.at[c, b], gsem)
            g.start()
            gathers.append(g)
    for g in gathers:
        g.wait()

    h1 = b1_ref[...][None, :]
    for c in range(CTX_LEN):
        h1 = h1 + jnp.dot(ebuf[c], w1_ref[c],
                          preferred_element_type=jnp.float32)
    h1 = jnp.maximum(h1, 0.0)
    h2 = jnp.maximum(
        jnp.dot(h1, w2_ref[...],
                preferred_element_type=jnp.float32) + b2_ref[...][None, :],
        0.0)

    for i in range(NUM_FULL_TILES):
        pltpu.make_async_copy(
            w3_hbm.at[:, pl.ds(i * VOCAB_TILE, VOCAB_TILE)],
            bufs.at[i % NBUF],
            sems.at[i % NBUF],
        ).wait()
        tile = jnp.dot(h2, bufs[i % NBUF],
                       preferred_element_type=jnp.float32)
        if i + NBUF < NUM_FULL_TILES:
            start_fetch(i + NBUF)
        out_ref[:, pl.ds(i * VOCAB_TILE, VOCAB_TILE)] = (
            tile + b3_ref[pl.ds(i * VOCAB_TILE, VOCAB_TILE)][None, :])

    tail_copy.wait()
    base = NUM_FULL_TILES * VOCAB_TILE
    tail = jnp.dot(h2, tail_buf[...], preferred_element_type=jnp.float32)
    out_ref[:, pl.ds(base, TAIL)] = tail + b3_ref[pl.ds(base, TAIL)][None, :]


def kernel(x, emb, W1, b1, W2, b2, W3, b3):
    return pl.pallas_call(
        _mlp_kernel,
        in_specs=[
            pl.BlockSpec(memory_space=pltpu.SMEM),
            pl.BlockSpec(memory_space=pl.ANY),
            pl.BlockSpec(memory_space=pltpu.VMEM),
            pl.BlockSpec(memory_space=pltpu.VMEM),
            pl.BlockSpec(memory_space=pltpu.VMEM),
            pl.BlockSpec(memory_space=pltpu.VMEM),
            pl.BlockSpec(memory_space=pl.ANY),
            pl.BlockSpec(memory_space=pltpu.VMEM),
        ],
        out_specs=pl.BlockSpec(memory_space=pltpu.VMEM),
        out_shape=jax.ShapeDtypeStruct((BATCH, VOCAB_SIZE), jnp.float32),
        scratch_shapes=[
            pltpu.VMEM((CTX_LEN, BATCH, EMB_DIM), jnp.float32),
            pltpu.VMEM((NBUF, H2, VOCAB_TILE), jnp.float32),
            pltpu.VMEM((H2, TAIL), jnp.float32),
            pltpu.SemaphoreType.DMA,
            pltpu.SemaphoreType.DMA((NBUF,)),
            pltpu.SemaphoreType.DMA,
        ],
    )(x, emb, W1.reshape(CTX_LEN, EMB_DIM, H1), b1, W2, b2, W3, b3)


# speedup vs baseline: 2.8543x; 1.0022x over previous
import jax
import jax.numpy as jnp
from jax.experimental import pallas as pl
from jax.experimental.pallas import tpu as pltpu

VOCAB_SIZE = 25107
EMB_DIM = 100
CTX_LEN = 5
BATCH = 16
H1 = 300
H2 = 300

VOCAB_TILE = 2048
NUM_FULL_TILES = VOCAB_SIZE // VOCAB_TILE
TAIL = VOCAB_SIZE - NUM_FULL_TILES * VOCAB_TILE
NBUF = 4


def _mlp_kernel(x_smem, emb_hbm, w1_ref, b1_ref, w2_ref, b2_ref, w3_hbm,
                b3_ref, out_ref, ebuf, bufs, tail_buf, gsem, sems, tail_sem):
    def start_fetch(i):
        pltpu.make_async_copy(
            w3_hbm.at[:, pl.ds(i * VOCAB_TILE, VOCAB_TILE)],
            bufs.at[i % NBUF],
            sems.at[i % NBUF],
        ).start()

    tail_copy = pltpu.make_async_copy(
        w3_hbm.at[:, pl.ds(NUM_FULL_TILES * VOCAB_TILE, TAIL)],
        tail_buf,
        tail_sem,
    )
    tail_copy.start()
    for i in range(NBUF):
        start_fetch(i)

    gathers = []
    for b in range(BATCH):
        for c in range(CTX_LEN):
            g = pltpu.make_async_copy(
                emb_hbm.at[pl.ds(x_smem[b, c], 1), :],
                ebuf.at[c, pl.ds(b, 1), :], gsem)
            g.start()
            gathers.append(g)
    for g in gathers:
        g.wait()

    h1 = b1_ref[...][None, :]
    for c in range(CTX_LEN):
        h1 = h1 + jnp.dot(ebuf[c], w1_ref[c],
                          preferred_element_type=jnp.float32)
    h1 = jnp.maximum(h1, 0.0)
    h2 = jnp.maximum(
        jnp.dot(h1, w2_ref[...],
                preferred_element_type=jnp.float32) + b2_ref[...][None, :],
        0.0)

    for i in range(NUM_FULL_TILES):
        pltpu.make_async_copy(
            w3_hbm.at[:, pl.ds(i * VOCAB_TILE, VOCAB_TILE)],
            bufs.at[i % NBUF],
            sems.at[i % NBUF],
        ).wait()
        tile = jnp.dot(h2, bufs[i % NBUF],
                       preferred_element_type=jnp.float32)
        if i + NBUF < NUM_FULL_TILES:
            start_fetch(i + NBUF)
        out_ref[:, pl.ds(i * VOCAB_TILE, VOCAB_TILE)] = (
            tile + b3_ref[pl.ds(i * VOCAB_TILE, VOCAB_TILE)][None, :])

    tail_copy.wait()
    base = NUM_FULL_TILES * VOCAB_TILE
    tail = jnp.dot(h2, tail_buf[...], preferred_element_type=jnp.float32)
    out_ref[:, pl.ds(base, TAIL)] = tail + b3_ref[pl.ds(base, TAIL)][None, :]


def kernel(x, emb, W1, b1, W2, b2, W3, b3):
    return pl.pallas_call(
        _mlp_kernel,
        in_specs=[
            pl.BlockSpec(memory_space=pltpu.SMEM),
            pl.BlockSpec(memory_space=pl.ANY),
            pl.BlockSpec(memory_space=pltpu.VMEM),
            pl.BlockSpec(memory_space=pltpu.VMEM),
            pl.BlockSpec(memory_space=pltpu.VMEM),
            pl.BlockSpec(memory_space=pltpu.VMEM),
            pl.BlockSpec(memory_space=pl.ANY),
            pl.BlockSpec(memory_space=pltpu.VMEM),
        ],
        out_specs=pl.BlockSpec(memory_space=pltpu.VMEM),
        out_shape=jax.ShapeDtypeStruct((BATCH, VOCAB_SIZE), jnp.float32),
        scratch_shapes=[
            pltpu.VMEM((CTX_LEN, BATCH, EMB_DIM), jnp.float32),
            pltpu.VMEM((NBUF, H2, VOCAB_TILE), jnp.float32),
            pltpu.VMEM((H2, TAIL), jnp.float32),
            pltpu.SemaphoreType.DMA,
            pltpu.SemaphoreType.DMA((NBUF,)),
            pltpu.SemaphoreType.DMA,
        ],
    )(x, emb, W1.reshape(CTX_LEN, EMB_DIM, H1), b1, W2, b2, W3, b3)
